# baseline (speedup 1.0000x reference)
import functools

import jax
import jax.numpy as jnp
from jax import lax
from jax.experimental import pallas as pl
from jax.experimental.pallas import tpu as pltpu
from jax.experimental.pallas import tpu_sc as plsc

TOPK = 8
NUM_EXPERTS = 64
HIDDEN = 2048
NC = 2
NS = 16
LANES = 16
NW = NC * NS
TOKENS = 8192
CHUNKS = 2
TOK_C = TOKENS // CHUNKS
TPW = TOK_C // NW
NG = TPW // LANES


def _score_block(x_ref, w_ref, b_ref, out_ref):
    logits = lax.dot_general(w_ref[...], x_ref[...], (((1,), (1,)), ((), ())),
                             preferred_element_type=jnp.float32)
    out_ref[0] = logits + b_ref[...]


def _scores_tc(x, weight, bias2d):
    return pl.pallas_call(
        _score_block,
        grid=(NW,),
        in_specs=[
            pl.BlockSpec((TPW, HIDDEN), lambda i: (i, 0)),
            pl.BlockSpec((NUM_EXPERTS, HIDDEN), lambda i: (0, 0)),
            pl.BlockSpec((NUM_EXPERTS, 1), lambda i: (0, 0)),
        ],
        out_specs=pl.BlockSpec((1, NUM_EXPERTS, TPW), lambda i: (i, 0, 0)),
        out_shape=jax.ShapeDtypeStruct((NW, NUM_EXPERTS, TPW), jnp.float32),
    )(x, weight, bias2d)


def _route_body(scores_hbm, bias_hbm, oidx_hbm, owgt_hbm,
                scores_v, bias_v, oidx_v, owgt_v):
    wid = lax.axis_index("s") * NC + lax.axis_index("c")
    pltpu.sync_copy(scores_hbm.at[wid], scores_v)
    pltpu.sync_copy(bias_hbm, bias_v)

    neg_inf = jnp.full((LANES,), -jnp.inf, jnp.float32)
    zero_i = jnp.zeros((LANES,), jnp.int32)
    b_regs = [bias_v[pl.ds(k * LANES, LANES)]
              for k in range(NUM_EXPERTS // LANES)]

    _gdn = lax.GatherDimensionNumbers(
        offset_dims=(), collapsed_slice_dims=(0,), start_index_map=(0,))

    def _reg_gather(vec, lane):
        return lax.gather(vec, lane[:, None], _gdn, slice_sizes=(1,),
                          mode=lax.GatherScatterMode.PROMISE_IN_BOUNDS)

    def _bias_at(i):
        lane = i & (LANES - 1)
        grp = i >> 4
        out = _reg_gather(b_regs[0], lane)
        for k in range(1, NUM_EXPERTS // LANES):
            out = jnp.where(grp == k, _reg_gather(b_regs[k], lane), out)
        return out

    def group(g, carry):
        base = g * LANES
        tops = [neg_inf] * TOPK
        idxs = [zero_i] * TOPK
        for e in range(NUM_EXPERTS):
            v = scores_v[e, pl.ds(base, LANES)]
            evec = jnp.full((LANES,), e, jnp.int32)
            cmp = [v > tops[j] for j in range(TOPK)]
            new_t = [jnp.where(cmp[0], v, tops[0])]
            new_i = [jnp.where(cmp[0], evec, idxs[0])]
            for j in range(1, TOPK):
                shift_t = jnp.where(cmp[j - 1], tops[j - 1], v)
                shift_i = jnp.where(cmp[j - 1], idxs[j - 1], evec)
                new_t.append(jnp.where(cmp[j], shift_t, tops[j]))
                new_i.append(jnp.where(cmp[j], shift_i, idxs[j]))
            tops, idxs = new_t, new_i
        probs = []
        for j in range(TOPK):
            raw = tops[j] - _bias_at(idxs[j])
            probs.append(1.0 / (1.0 + jnp.exp(-raw)))
        denom = jnp.abs(probs[0])
        for j in range(1, TOPK):
            denom = denom + jnp.abs(probs[j])
        inv = 1.0 / jnp.maximum(denom, 1e-12)
        for j in range(TOPK):
            oidx_v[j, pl.ds(base, LANES)] = idxs[j]
            owgt_v[j, pl.ds(base, LANES)] = probs[j] * inv
        return carry

    lax.fori_loop(0, NG, group, 0)
    pltpu.sync_copy(oidx_v, oidx_hbm.at[wid])
    pltpu.sync_copy(owgt_v, owgt_hbm.at[wid])


def _route_sc(scores, expert_biases):
    mesh = plsc.VectorSubcoreMesh(core_axis_name="c", subcore_axis_name="s")
    return pl.kernel(
        _route_body,
        out_type=(
            jax.ShapeDtypeStruct((NW, TOPK, TPW), jnp.int32),
            jax.ShapeDtypeStruct((NW, TOPK, TPW), jnp.float32),
        ),
        mesh=mesh,
        scratch_types=[
            pltpu.VMEM((NUM_EXPERTS, TPW), jnp.float32),
            pltpu.VMEM((NUM_EXPERTS,), jnp.float32),
            pltpu.VMEM((TOPK, TPW), jnp.int32),
            pltpu.VMEM((TOPK, TPW), jnp.float32),
        ],
    )(scores, expert_biases)


@jax.jit
def _gate(x, weight, bias2d, bias1d):
    idxs = []
    wgts = []
    for c in range(CHUNKS):
        xc = lax.slice_in_dim(x, c * TOK_C, (c + 1) * TOK_C, axis=0)
        scores = _scores_tc(xc, weight, bias2d)
        idx_t, wgt_t = _route_sc(scores, bias1d)
        idxs.append(idx_t.transpose(0, 2, 1).reshape(TOK_C, TOPK))
        wgts.append(wgt_t.transpose(0, 2, 1).reshape(TOK_C, TOPK))
    return jnp.concatenate(idxs, axis=0), jnp.concatenate(wgts, axis=0)


def kernel(hidden_states, weight, expert_biases):
    bsz, seq_len, h = hidden_states.shape
    x = hidden_states.reshape(-1, h)
    idx_flat, wgt_flat = _gate(x, weight,
                               expert_biases.reshape(NUM_EXPERTS, 1),
                               expert_biases)
    return (idx_flat.reshape(bsz, seq_len, TOPK),
            wgt_flat.reshape(bsz, seq_len, TOPK))


# speedup vs baseline: 1.9940x; 1.9940x over previous
import functools

import jax
import jax.numpy as jnp
from jax import lax
from jax.experimental import pallas as pl
from jax.experimental.pallas import tpu as pltpu
from jax.experimental.pallas import tpu_sc as plsc

TOPK = 8
NUM_EXPERTS = 64
HIDDEN = 2048
NC = 2
NS = 16
LANES = 16
NW = NC * NS
TOKENS = 8192
TPW = TOKENS // NW
NG = TPW // LANES


def _score_block(x_ref, w_ref, b_ref, out_ref):
    logits = lax.dot_general(w_ref[...], x_ref[...], (((1,), (1,)), ((), ())),
                             preferred_element_type=jnp.float32)
    out_ref[0] = logits + b_ref[...]


def _scores_tc(x, weight, bias2d):
    return pl.pallas_call(
        _score_block,
        grid=(NW,),
        in_specs=[
            pl.BlockSpec((TPW, HIDDEN), lambda i: (i, 0)),
            pl.BlockSpec((NUM_EXPERTS, HIDDEN), lambda i: (0, 0)),
            pl.BlockSpec((NUM_EXPERTS, 1), lambda i: (0, 0)),
        ],
        out_specs=pl.BlockSpec((1, NUM_EXPERTS, TPW), lambda i: (i, 0, 0)),
        out_shape=jax.ShapeDtypeStruct((NW, NUM_EXPERTS, TPW), jnp.float32),
    )(x, weight, bias2d)


def _route_body(scores_hbm, bias_hbm, oidx_hbm, owgt_hbm,
                scores_v, bias_v, oidx_v, owgt_v, sem_s, sem_b):
    wid = lax.axis_index("s") * NC + lax.axis_index("c")
    cp_s = pltpu.make_async_copy(scores_hbm.at[wid], scores_v, sem_s)
    cp_b = pltpu.make_async_copy(bias_hbm, bias_v, sem_b)
    cp_s.start()
    cp_b.start()
    cp_s.wait()
    cp_b.wait()

    lane_iota = lax.iota(jnp.int32, LANES)
    neg_inf = jnp.full((LANES,), -jnp.inf, jnp.float32)
    zero_i = jnp.zeros((LANES,), jnp.int32)
    b_regs = [bias_v[pl.ds(k * LANES, LANES)] for k in range(NUM_EXPERTS // LANES)]

    _gdn = lax.GatherDimensionNumbers(
        offset_dims=(), collapsed_slice_dims=(0,), start_index_map=(0,))

    def _reg_gather(vec, lane):
        return lax.gather(vec, lane[:, None], _gdn, slice_sizes=(1,),
                          mode=lax.GatherScatterMode.PROMISE_IN_BOUNDS)

    def _bias_at(i):
        lane = i & (LANES - 1)
        grp = i >> 4
        out = _reg_gather(b_regs[0], lane)
        for k in range(1, NUM_EXPERTS // LANES):
            out = jnp.where(grp == k, _reg_gather(b_regs[k], lane), out)
        return out

    def group(g, carry):
        base = g * LANES
        tops = [neg_inf] * TOPK
        idxs = [zero_i] * TOPK
        for e in range(NUM_EXPERTS):
            v = scores_v[e, pl.ds(base, LANES)]
            evec = jnp.full((LANES,), e, jnp.int32)
            cmp = [v > tops[j] for j in range(TOPK)]
            new_t = [jnp.where(cmp[0], v, tops[0])]
            new_i = [jnp.where(cmp[0], evec, idxs[0])]
            for j in range(1, TOPK):
                shift_t = jnp.where(cmp[j - 1], tops[j - 1], v)
                shift_i = jnp.where(cmp[j - 1], idxs[j - 1], evec)
                new_t.append(jnp.where(cmp[j], shift_t, tops[j]))
                new_i.append(jnp.where(cmp[j], shift_i, idxs[j]))
            tops, idxs = new_t, new_i
        probs = []
        for j in range(TOPK):
            raw = tops[j] - _bias_at(idxs[j])
            probs.append(1.0 / (1.0 + jnp.exp(-raw)))
        denom = probs[0]
        for j in range(1, TOPK):
            denom = denom + probs[j]
        inv = 1.0 / jnp.maximum(denom, 1e-12)
        for j in range(TOPK):
            oidx_v[j, pl.ds(base, LANES)] = idxs[j]
            owgt_v[j, pl.ds(base, LANES)] = probs[j] * inv
        return carry

    lax.fori_loop(0, NG, group, 0)
    pltpu.sync_copy(oidx_v, oidx_hbm.at[wid])
    pltpu.sync_copy(owgt_v, owgt_hbm.at[wid])


def _route_sc(scores, expert_biases):
    mesh = plsc.VectorSubcoreMesh(core_axis_name="c", subcore_axis_name="s")
    return pl.kernel(
        _route_body,
        out_type=(
            jax.ShapeDtypeStruct((NW, TOPK, TPW), jnp.int32),
            jax.ShapeDtypeStruct((NW, TOPK, TPW), jnp.float32),
        ),
        mesh=mesh,
        scratch_types=[
            pltpu.VMEM((NUM_EXPERTS, TPW), jnp.float32),
            pltpu.VMEM((NUM_EXPERTS,), jnp.float32),
            pltpu.VMEM((TOPK, TPW), jnp.int32),
            pltpu.VMEM((TOPK, TPW), jnp.float32),
            pltpu.SemaphoreType.DMA,
            pltpu.SemaphoreType.DMA,
        ],
    )(scores, expert_biases)


@jax.jit
def _gate(x, weight, bias2d, bias1d):
    scores = _scores_tc(x, weight, bias2d)
    idx_t, wgt_t = _route_sc(scores, bias1d)
    idx_flat = idx_t.transpose(0, 2, 1).reshape(TOKENS, TOPK)
    wgt_flat = wgt_t.transpose(0, 2, 1).reshape(TOKENS, TOPK)
    return idx_flat, wgt_flat


def kernel(hidden_states, weight, expert_biases):
    bsz, seq_len, h = hidden_states.shape
    x = hidden_states.reshape(-1, h)
    idx_flat, wgt_flat = _gate(x, weight,
                               expert_biases.reshape(NUM_EXPERTS, 1),
                               expert_biases)
    return (idx_flat.reshape(bsz, seq_len, TOPK),
            wgt_flat.reshape(bsz, seq_len, TOPK))


# speedup vs baseline: 2.0009x; 1.0034x over previous
import jax
import jax.numpy as jnp
from jax import lax
from jax.experimental import pallas as pl
from jax.experimental.pallas import tpu as pltpu
from jax.experimental.pallas import tpu_sc as plsc

TOPK = 8
NUM_EXPERTS = 64
HIDDEN = 2048
NC = 2
NS = 16
LANES = 16
NW = NC * NS
TOKENS = 8192
TPW = TOKENS // NW
NG = TPW // LANES


def _score_block(x_ref, w_ref, b_ref, out_ref):
    logits = lax.dot_general(w_ref[...], x_ref[...], (((1,), (1,)), ((), ())),
                             preferred_element_type=jnp.float32)
    out_ref[0] = logits + b_ref[...]


def _scores_tc(x, weight, bias2d):
    return pl.pallas_call(
        _score_block,
        grid=(NW,),
        in_specs=[
            pl.BlockSpec((TPW, HIDDEN), lambda i: (i, 0)),
            pl.BlockSpec((NUM_EXPERTS, HIDDEN), lambda i: (0, 0)),
            pl.BlockSpec((NUM_EXPERTS, 1), lambda i: (0, 0)),
        ],
        out_specs=pl.BlockSpec((1, NUM_EXPERTS, TPW), lambda i: (i, 0, 0)),
        out_shape=jax.ShapeDtypeStruct((NW, NUM_EXPERTS, TPW), jnp.float32),
    )(x, weight, bias2d)


def _route_body(scores_hbm, bias_hbm, oidx_hbm, owgt_hbm,
                scores_v, bias_v, oidx_v, owgt_v, sem_s, sem_b):
    wid = lax.axis_index("s") * NC + lax.axis_index("c")
    cp_s = pltpu.make_async_copy(scores_hbm.at[wid], scores_v, sem_s)
    cp_b = pltpu.make_async_copy(bias_hbm, bias_v, sem_b)
    cp_s.start()
    cp_b.start()
    cp_s.wait()
    cp_b.wait()

    lane_iota = lax.iota(jnp.int32, LANES)
    neg_inf = jnp.full((LANES,), -jnp.inf, jnp.float32)
    zero_i = jnp.zeros((LANES,), jnp.int32)
    b_regs = [bias_v[pl.ds(k * LANES, LANES)] for k in range(NUM_EXPERTS // LANES)]

    _gdn = lax.GatherDimensionNumbers(
        offset_dims=(), collapsed_slice_dims=(0,), start_index_map=(0,))

    def _reg_gather(vec, lane):
        return lax.gather(vec, lane[:, None], _gdn, slice_sizes=(1,),
                          mode=lax.GatherScatterMode.PROMISE_IN_BOUNDS)

    def _bias_at(i):
        lane = i & (LANES - 1)
        grp = i >> 4
        out = _reg_gather(b_regs[0], lane)
        for k in range(1, NUM_EXPERTS // LANES):
            out = jnp.where(grp == k, _reg_gather(b_regs[k], lane), out)
        return out

    def group(g, carry):
        base = g * LANES
        tops = [neg_inf] * TOPK
        idxs = [zero_i] * TOPK
        for e in range(NUM_EXPERTS):
            v = scores_v[e, pl.ds(base, LANES)]
            evec = jnp.full((LANES,), e, jnp.int32)
            cmp = [v > tops[j] for j in range(TOPK)]
            new_t = [jnp.where(cmp[0], v, tops[0])]
            new_i = [jnp.where(cmp[0], evec, idxs[0])]
            for j in range(1, TOPK):
                shift_t = jnp.where(cmp[j - 1], tops[j - 1], v)
                shift_i = jnp.where(cmp[j - 1], idxs[j - 1], evec)
                new_t.append(jnp.where(cmp[j], shift_t, tops[j]))
                new_i.append(jnp.where(cmp[j], shift_i, idxs[j]))
            tops, idxs = new_t, new_i
        probs = []
        for j in range(TOPK):
            raw = tops[j] - _bias_at(idxs[j])
            probs.append(1.0 / (1.0 + jnp.exp(-raw)))
        denom = probs[0]
        for j in range(1, TOPK):
            denom = denom + probs[j]
        inv = 1.0 / jnp.maximum(denom, 1e-12)
        for j in range(TOPK):
            oidx_v[j, pl.ds(base, LANES)] = idxs[j]
            owgt_v[j, pl.ds(base, LANES)] = probs[j] * inv
        return carry

    lax.fori_loop(0, NG, group, 0)
    pltpu.sync_copy(oidx_v, oidx_hbm.at[wid])
    pltpu.sync_copy(owgt_v, owgt_hbm.at[wid])


def _route_sc(scores, expert_biases):
    mesh = plsc.VectorSubcoreMesh(core_axis_name="c", subcore_axis_name="s")
    return pl.kernel(
        _route_body,
        out_type=(
            jax.ShapeDtypeStruct((NW, TOPK, TPW), jnp.int32),
            jax.ShapeDtypeStruct((NW, TOPK, TPW), jnp.float32),
        ),
        mesh=mesh,
        scratch_types=[
            pltpu.VMEM((NUM_EXPERTS, TPW), jnp.float32),
            pltpu.VMEM((NUM_EXPERTS,), jnp.float32),
            pltpu.VMEM((TOPK, TPW), jnp.int32),
            pltpu.VMEM((TOPK, TPW), jnp.float32),
            pltpu.SemaphoreType.DMA,
            pltpu.SemaphoreType.DMA,
        ],
    )(scores, expert_biases)


@jax.jit
def _gate(x, weight, bias2d, bias1d):
    scores = _scores_tc(x, weight, bias2d)
    idx_t, wgt_t = _route_sc(scores, bias1d)
    idx_flat = idx_t.transpose(0, 2, 1).reshape(TOKENS, TOPK)
    wgt_flat = wgt_t.transpose(0, 2, 1).reshape(TOKENS, TOPK)
    return idx_flat, wgt_flat


def kernel(hidden_states, weight, expert_biases):
    bsz, seq_len, h = hidden_states.shape
    x = hidden_states.reshape(-1, h)
    idx_flat, wgt_flat = _gate(x, weight,
                               expert_biases.reshape(NUM_EXPERTS, 1),
                               expert_biases)
    return (idx_flat.reshape(bsz, seq_len, TOPK),
            wgt_flat.reshape(bsz, seq_len, TOPK))
